# Initial kernel scaffold; baseline (speedup 1.0000x reference)
import jax
import jax.numpy as jnp
from jax.experimental import pallas as pl


def kernel(anonymized_nodes, walk_masks, kernel):
    raise NotImplementedError("write your pallas kernel here")


# speedup vs baseline: 1587.3981x; 1587.3981x over previous
import functools

import jax
import jax.numpy as jnp
from jax.experimental import pallas as pl
from jax.experimental.pallas import tpu as pltpu

_INTERPRET = False


def _cooc_kernel(nodes_ref, nodesT_ref, mask_ref, maskT_ref, ker_ref, out_ref):
    nodes = nodes_ref[0]
    nodesT = nodesT_ref[0]
    mask = mask_ref[0]
    maskT = maskT_ref[0]
    ker = ker_ref[...]
    W, L = nodes.shape

    wi = jax.lax.broadcasted_iota(jnp.int32, (W, L), 0)
    pi = jax.lax.broadcasted_iota(jnp.int32, (W, L), 1)
    nm = jnp.where(mask != 0.0, nodes, -1 - (wi * L + pi))
    wiT = jax.lax.broadcasted_iota(jnp.int32, (L, W), 1)
    piT = jax.lax.broadcasted_iota(jnp.int32, (L, W), 0)
    nmT = jnp.where(maskT != 0.0, nodesT, -1 - (wiT * L + piT))

    acc = jnp.zeros((W, W), jnp.float32)
    corr = jnp.zeros((W, 1), jnp.float32)
    for p1 in range(L):
        n1 = jax.lax.slice(nm, (0, p1), (W, p1 + 1))
        esum = jnp.zeros((W, W), jnp.float32)
        for p2 in range(L):
            n2 = jax.lax.slice(nmT, (p2, 0), (p2 + 1, W))
            eq = n1 == n2
            acc = acc + jnp.where(eq, ker[p1, p2], 0.0)
            esum = esum + jnp.where(eq, 1.0, 0.0)
        cnt = jnp.sum(esum, axis=1, keepdims=True)
        corr = corr + jnp.where(cnt == 1.0, ker[p1, p1], 0.0)

    ri = jax.lax.broadcasted_iota(jnp.int32, (W, W), 0)
    ci = jax.lax.broadcasted_iota(jnp.int32, (W, W), 1)
    acc = acc - jnp.where(ri == ci, corr, 0.0)

    lens_c = jnp.sum(mask, axis=1, keepdims=True)
    lens_r = jnp.sum(maskT, axis=0, keepdims=True)
    norm = jnp.maximum(lens_c * lens_r, 1e-6)
    valid = (lens_c > 0.0) & (lens_r > 0.0)
    co = jnp.where(valid, acc / norm, 0.0)
    co = jnp.clip(co, -10.0, 10.0)
    out_ref[0] = jnp.tanh(co)


def kernel(anonymized_nodes, walk_masks, kernel):
    B, W, L = anonymized_nodes.shape
    nodesT = jnp.swapaxes(anonymized_nodes, 1, 2)
    maskT = jnp.swapaxes(walk_masks, 1, 2)
    out = pl.pallas_call(
        _cooc_kernel,
        grid=(B,),
        in_specs=[
            pl.BlockSpec((1, W, L), lambda b: (b, 0, 0)),
            pl.BlockSpec((1, L, W), lambda b: (b, 0, 0)),
            pl.BlockSpec((1, W, L), lambda b: (b, 0, 0)),
            pl.BlockSpec((1, L, W), lambda b: (b, 0, 0)),
            pl.BlockSpec((L, L), lambda b: (0, 0)),
        ],
        out_specs=pl.BlockSpec((1, W, W), lambda b: (b, 0, 0)),
        out_shape=jax.ShapeDtypeStruct((B, W, W), jnp.float32),
        compiler_params=pltpu.CompilerParams(
            dimension_semantics=("arbitrary",),
        ),
        interpret=_INTERPRET,
    )(anonymized_nodes, nodesT, walk_masks, maskT, kernel)
    return out
